# Initial kernel scaffold; baseline (speedup 1.0000x reference)
import jax
import jax.numpy as jnp
from jax.experimental import pallas as pl


def kernel(seq, table):
    raise NotImplementedError("write your pallas kernel here")


# speedup vs baseline: 1.3756x; 1.3756x over previous
import jax
import jax.numpy as jnp
from jax import lax
from jax.experimental import pallas as pl
from jax.experimental.pallas import tpu as pltpu
from jax.experimental.pallas import tpu_sc as plsc

B, L, V, D = 1024, 200, 6, 128
NC, NS, LANES = 2, 16, 16
NW = NC * NS
BPW = B // NW
NVEC = (L + LANES - 1) // LANES
TAIL = L - (NVEC - 1) * LANES


def _body(seq_hbm, tbl_hbm, out_hbm, seq_v, tbl_v, blk_v):
    wid = lax.axis_index("s") * NC + lax.axis_index("c")
    base_b = wid * BPW
    pltpu.sync_copy(seq_hbm.at[pl.ds(base_b * L, BPW * L)], seq_v)
    pltpu.sync_copy(tbl_hbm, tbl_v)

    lanes = jnp.arange(LANES, dtype=jnp.int32)
    tail_mask = lanes < TAIL

    def per_b(bi, carry):
        for j in range(NVEC):
            li = jnp.minimum(j * LANES + lanes, L - 1)
            seqv = plsc.load_gather(seq_v, [bi * L + li])
            addr0 = seqv * D

            if j < NVEC - 1:
                @plsc.parallel_loop(0, D, step=16)
                def _dloop(d0):
                    for k in range(16):
                        val = plsc.load_gather(tbl_v, [addr0 + (d0 + k)])
                        blk_v[d0 + k, pl.ds(j * LANES, LANES)] = val
            else:
                @plsc.parallel_loop(0, D, step=16)
                def _dtail(d0):
                    for k in range(16):
                        val = plsc.load_gather(tbl_v, [addr0 + (d0 + k)])
                        plsc.store_scatter(
                            blk_v,
                            [jnp.full((LANES,), d0 + k, jnp.int32), li],
                            val,
                            mask=tail_mask,
                        )
        pltpu.sync_copy(blk_v, out_hbm.at[base_b + bi])
        return carry

    lax.fori_loop(0, BPW, per_b, jnp.int32(0))


@jax.jit
def kernel(seq, table):
    seq_flat = seq.astype(jnp.int32).reshape(B * L)
    tbl_flat = table.reshape(V * D)
    run = pl.kernel(
        _body,
        out_type=jax.ShapeDtypeStruct((B, D, L), jnp.float32),
        mesh=plsc.VectorSubcoreMesh(core_axis_name="c", subcore_axis_name="s"),
        compiler_params=pltpu.CompilerParams(needs_layout_passes=False),
        scratch_types=[
            pltpu.VMEM((BPW * L,), jnp.int32),
            pltpu.VMEM((V * D,), jnp.float32),
            pltpu.VMEM((D, L), jnp.float32),
        ],
    )
    return run(seq_flat, tbl_flat)
